# baseline (speedup 1.0000x reference)
import jax
import jax.numpy as jnp
from jax import lax
from jax.experimental import pallas as pl
from jax.experimental.pallas import tpu as pltpu
from jax.experimental.pallas import tpu_sc as plsc

_BATCH = 1024
_SEQ = 200
_D = 64
_NC = 2
_NS = 16
_NW = _NC * _NS
_RPW = _BATCH // _NW
_NBUF = 2


def _sc_kernel_body(x_hbm, table_hbm, pe_hbm, out_hbm,
                    idx0_v, idx1_v, rows_v, outb_v, pe_v,
                    isem, gsem, osem):
    wid = lax.axis_index("subcore") * _NC + lax.axis_index("core")
    base = wid * _RPW
    idx_bufs = (idx0_v, idx1_v)

    pltpu.sync_copy(pe_hbm, pe_v)

    def idx_start(b):
        p = b % _NBUF
        pltpu.make_async_copy(
            x_hbm.at[base + b], idx_bufs[p], isem.at[p]).start()

    def idx_wait(b):
        p = b % _NBUF
        pltpu.make_async_copy(
            x_hbm.at[base + b], idx_bufs[p], isem.at[p]).wait()

    def gather_start(b):
        p = b % _NBUF
        pltpu.make_async_copy(
            table_hbm.at[idx_bufs[p]], rows_v.at[p], gsem.at[p]).start()

    def gather_wait(b):
        p = b % _NBUF
        pltpu.make_async_copy(
            table_hbm.at[idx_bufs[p]], rows_v.at[p], gsem.at[p]).wait()

    def out_start(b):
        p = b % _NBUF
        pltpu.make_async_copy(
            outb_v.at[p], out_hbm.at[base + b], osem.at[p]).start()

    def out_wait(b):
        p = b % _NBUF
        pltpu.make_async_copy(
            outb_v.at[p], out_hbm.at[base + b], osem.at[p]).wait()

    for b in range(_NBUF):
        idx_start(b)
        idx_wait(b)
        gather_start(b)

    for b in range(_RPW):
        p = b % _NBUF
        gather_wait(b)
        if b + _NBUF < _RPW:
            idx_start(b + _NBUF)
        if b >= _NBUF:
            out_wait(b - _NBUF)

        @pl.loop(0, _SEQ)
        def _(r, p=p):
            for c in range(0, _D, 16):
                outb_v[p, r, pl.ds(c, 16)] = (
                    rows_v[p, r, pl.ds(c, 16)] + pe_v[r, pl.ds(c, 16)])

        if b + _NBUF < _RPW:
            idx_wait(b + _NBUF)
            gather_start(b + _NBUF)
        out_start(b)

    for b in range(_RPW - _NBUF, _RPW):
        out_wait(b)


@jax.jit
def _positional_embedding_sc(x, table, pe_tile):
    mesh = plsc.VectorSubcoreMesh(
        core_axis_name="core", subcore_axis_name="subcore")
    kern = pl.kernel(
        _sc_kernel_body,
        out_type=jax.ShapeDtypeStruct((_BATCH, _SEQ, _D), jnp.float32),
        mesh=mesh,
        compiler_params=pltpu.CompilerParams(use_tc_tiling_on_sc=False),
        scratch_types=[
            pltpu.VMEM((_SEQ,), jnp.int32),
            pltpu.VMEM((_SEQ,), jnp.int32),
            pltpu.VMEM((_NBUF, _SEQ, _D), jnp.float32),
            pltpu.VMEM((_NBUF, _SEQ, _D), jnp.float32),
            pltpu.VMEM((_SEQ, _D), jnp.float32),
            pltpu.SemaphoreType.DMA((_NBUF,)),
            pltpu.SemaphoreType.DMA((_NBUF,)),
            pltpu.SemaphoreType.DMA((_NBUF,)),
        ],
    )
    return kern(x, table, pe_tile)


def kernel(X, table, pe):
    seq_len = X.shape[-1]
    pe_tile = pe[0, :seq_len, :]
    return _positional_embedding_sc(X.astype(jnp.int32), table, pe_tile)
